# baseline (speedup 1.0000x reference)
import functools

import jax
import jax.numpy as jnp
from jax import lax
from jax.experimental import pallas as pl
from jax.experimental.pallas import tpu as pltpu
from jax.experimental.pallas import tpu_sc as plsc


def _sc_gather(table, idx):
    nrows = idx.shape[0]
    d = table.shape[1]
    info = plsc.get_sparse_core_info()
    nw = info.num_cores * info.num_subcores
    rpw = nrows // nw
    mesh = plsc.VectorSubcoreMesh(core_axis_name="c", subcore_axis_name="s")

    @functools.partial(
        pl.kernel,
        mesh=mesh,
        out_type=jax.ShapeDtypeStruct((nrows, d), jnp.float32),
        scratch_types=[
            pltpu.VMEM((rpw,), jnp.int32),
            pltpu.VMEM((rpw, d), jnp.float32),
            pltpu.SemaphoreType.DMA,
        ],
    )
    def gk(table_hbm, idx_hbm, out_hbm, idx_v, rows_v, sem):
        wid = lax.axis_index("s") * info.num_cores + lax.axis_index("c")
        base = wid * rpw
        pltpu.sync_copy(idx_hbm.at[pl.ds(base, rpw)], idx_v)
        pltpu.async_copy(table_hbm.at[idx_v], rows_v, sem).wait()
        pltpu.sync_copy(rows_v, out_hbm.at[pl.ds(base, rpw)])

    return gk(table, idx)


def _tc_interact(wide3, par2, weff, lin_w, bias2, b, d, gb):
    f = wide3.shape[0]
    d2 = 2 * d

    def body(w_ref, p_ref, c_ref, l_ref, b_ref, o_ref):
        w3 = w_ref[...]
        half = lax.broadcasted_iota(jnp.int32, (f, gb, d2), 2) // d
        p3 = p_ref[...][:, :, None]
        ew = jnp.where(half == p3, w3, 0.0)
        pw2 = jnp.dot(c_ref[...], ew.reshape(f, gb * d2),
                      preferred_element_type=jnp.float32)
        pw3 = pw2.reshape(f, gb, d2)
        e_sel = ew[:, :, :d] + ew[:, :, d:]
        p_sel = pw3[:, :, :d] + pw3[:, :, d:]
        t3 = e_sel * (p_sel + l_ref[...][:, None, :])
        t2 = jnp.sum(t3, axis=0)
        s = jnp.sum(t2, axis=1, keepdims=True)
        o_ref[...] = jax.nn.sigmoid(s + b_ref[0, 0])

    return pl.pallas_call(
        body,
        grid=(b // gb,),
        in_specs=[
            pl.BlockSpec((f, gb, d2), lambda i: (0, i, 0)),
            pl.BlockSpec((f, gb), lambda i: (0, i)),
            pl.BlockSpec((f, f), lambda i: (0, 0)),
            pl.BlockSpec((f, d), lambda i: (0, 0)),
            pl.BlockSpec((1, 1), lambda i: (0, 0)),
        ],
        out_specs=pl.BlockSpec((gb, 1), lambda i: (i, 0)),
        out_shape=jax.ShapeDtypeStruct((b, 1), jnp.float32),
    )(wide3, par2, weff, lin_w, bias2)


def kernel(x, emb_tables, field_cov_w, lin_w, bias):
    b, f = x.shape
    _, v, d = emb_tables.shape
    gb = 256

    xt = x.T.astype(jnp.int32)
    rows = xt + (jnp.arange(f, dtype=jnp.int32) * v)[:, None]
    widx = (rows >> 1).reshape(-1)
    par2 = rows & 1
    table2 = emb_tables.reshape(f * v // 2, 2 * d)
    sym = (field_cov_w + field_cov_w.T) * 0.5
    weff = 0.5 * (sym - jnp.diag(jnp.diag(sym)))

    wide = _sc_gather(table2, widx)
    wide3 = wide.reshape(f, b, 2 * d)
    out = _tc_interact(wide3, par2, weff, lin_w, bias.reshape(1, 1), b, d, gb)
    return out.reshape(b)


# speedup vs baseline: 1.0125x; 1.0125x over previous
import functools

import jax
import jax.numpy as jnp
from jax import lax
from jax.experimental import pallas as pl
from jax.experimental.pallas import tpu as pltpu
from jax.experimental.pallas import tpu_sc as plsc


def _sc_gather(table, idx):
    nrows = idx.shape[0]
    d = table.shape[1]
    info = plsc.get_sparse_core_info()
    nw = info.num_cores * info.num_subcores
    rpw = nrows // nw
    mesh = plsc.VectorSubcoreMesh(core_axis_name="c", subcore_axis_name="s")

    @functools.partial(
        pl.kernel,
        mesh=mesh,
        out_type=jax.ShapeDtypeStruct((nrows, d), jnp.float32),
        scratch_types=[
            pltpu.VMEM((rpw,), jnp.int32),
            pltpu.VMEM((rpw, d), jnp.float32),
            pltpu.SemaphoreType.DMA,
        ],
        compiler_params=pltpu.CompilerParams(use_tc_tiling_on_sc=False),
    )
    def gk(table_hbm, idx_hbm, out_hbm, idx_v, rows_v, sem):
        wid = lax.axis_index("s") * info.num_cores + lax.axis_index("c")
        base = wid * rpw
        pltpu.sync_copy(idx_hbm.at[pl.ds(base, rpw)], idx_v)
        pltpu.async_copy(table_hbm.at[idx_v], rows_v, sem).wait()
        pltpu.sync_copy(rows_v, out_hbm.at[pl.ds(base, rpw)])

    return gk(table, idx)


def _tc_interact(e2, weff, lin_tile, gmat, bias2, b, d, gb):
    f, n = e2.shape
    nb = gb * d

    def body(e_ref, w_ref, lt_ref, g_ref, b_ref, o_ref):
        e = e_ref[...]
        p = jnp.dot(w_ref[...], e, preferred_element_type=jnp.float32)
        colsum = jnp.sum(e * (p + lt_ref[...]), axis=0, keepdims=True)
        red = jnp.dot(colsum, g_ref[...], preferred_element_type=jnp.float32)
        o_ref[...] = jax.nn.sigmoid(red + b_ref[0, 0])

    return pl.pallas_call(
        body,
        grid=(n // nb,),
        in_specs=[
            pl.BlockSpec((f, nb), lambda i: (0, i)),
            pl.BlockSpec((f, f), lambda i: (0, 0)),
            pl.BlockSpec((f, nb), lambda i: (0, 0)),
            pl.BlockSpec((nb, gb), lambda i: (0, 0)),
            pl.BlockSpec((1, 1), lambda i: (0, 0)),
        ],
        out_specs=pl.BlockSpec((1, gb), lambda i: (0, i)),
        out_shape=jax.ShapeDtypeStruct((1, b), jnp.float32),
    )(e2, weff, lin_tile, gmat, bias2)


def kernel(x, emb_tables, field_cov_w, lin_w, bias):
    b, f = x.shape
    _, v, d = emb_tables.shape
    gb = 128

    xt = x.T.astype(jnp.int32)
    offs = (jnp.arange(f, dtype=jnp.int32) * v)[:, None]
    idx = (xt + offs).reshape(-1)
    table = emb_tables.reshape(f * v, d)
    sym = (field_cov_w + field_cov_w.T) * 0.5
    weff = 0.5 * (sym - jnp.diag(jnp.diag(sym)))
    lin_tile = jnp.tile(lin_w, (1, gb))
    nb = gb * d
    gmat = (jnp.arange(nb, dtype=jnp.int32)[:, None] // d
            == jnp.arange(gb, dtype=jnp.int32)[None, :]).astype(jnp.float32)

    e_flat = _sc_gather(table, idx)
    e2 = e_flat.reshape(f, b * d)
    out2 = _tc_interact(e2, weff, lin_tile, gmat, bias.reshape(1, 1), b, d, gb)
    return out2.reshape(b)
